# baseline (speedup 1.0000x reference)
import functools

import jax
import jax.numpy as jnp
from jax import lax
from jax.experimental import pallas as pl
from jax.experimental.pallas import tpu as pltpu
from jax.experimental.pallas import tpu_sc as plsc

_DIM = 32
_NC = 2
_NS = 16
_NW = _NC * _NS
_L = 16

_STRIP = 31232
_RCHUNK = 256
_TAIL = 576


def _repack_body(u_hbm, v_hbm, u2_hbm, v2_hbm,
                 in0, out0, in1, out1, sem0, sem1):
    wid = lax.axis_index("s") * _NC + lax.axis_index("c")
    base = pl.multiple_of(wid * _STRIP, 256)
    nchunks = _STRIP // _RCHUNK

    def depad(src, dst, nrows):
        def row4(j, _):
            for k in range(4):
                dst[j, pl.ds(k * _DIM, _L)] = src[j * 4 + k, pl.ds(0, _L)]
                dst[j, pl.ds(k * _DIM + _L, _L)] = src[j * 4 + k, pl.ds(_L, _L)]
            return 0

        lax.fori_loop(0, nrows // 4, row4, 0)

    def start(t_hbm, row0, nrows, ibuf, sem):
        pltpu.async_copy(t_hbm.at[pl.ds(row0, nrows), :],
                         ibuf.at[pl.ds(0, nrows), :], sem)

    def finish(t_hbm, t2_hbm, row0, nrows, ibuf, obuf, sem):
        pltpu.make_async_copy(
            t_hbm.at[pl.ds(0, nrows), :], ibuf.at[pl.ds(0, nrows), :],
            sem).wait()
        depad(ibuf, obuf, nrows)
        orow = pl.multiple_of(lax.div(row0, 4), 16)
        pltpu.sync_copy(
            obuf.at[pl.ds(0, nrows // 4), :],
            t2_hbm.at[pl.ds(orow, nrows // 4), :])

    def one_table(t_hbm, t2_hbm):
        def off(g):
            return pl.multiple_of(base + g * _RCHUNK, 256)

        start(t_hbm, off(0), _RCHUNK, in0, sem0)

        def outer(t, _):
            g = t * 2
            start(t_hbm, off(g + 1), _RCHUNK, in1, sem1)
            finish(t_hbm, t2_hbm, off(g), _RCHUNK, in0, out0, sem0)

            @pl.when(g + 2 < nchunks)
            def _():
                start(t_hbm, off(g + 2), _RCHUNK, in0, sem0)

            finish(t_hbm, t2_hbm, off(g + 1), _RCHUNK, in1, out1, sem1)
            return 0

        lax.fori_loop(0, nchunks // 2, outer, 0)

    one_table(u_hbm, u2_hbm)
    one_table(v_hbm, v2_hbm)

    @pl.when(wid == _NW - 1)
    def _():
        tb = _NW * _STRIP

        def tail(t_hbm, t2_hbm):
            for q, sz in ((0, 256), (256, 256), (512, 64)):
                start(t_hbm, tb + q, sz, in0, sem0)
                finish(t_hbm, t2_hbm, tb + q, sz, in0, out0, sem0)

        tail(u_hbm, u2_hbm)
        tail(v_hbm, v2_hbm)


def _gather_body(tok_hbm, head_hbm, u2_hbm, v2_hbm, ub_hbm, vb_hbm, out_hbm,
                 tok_v, head_v, it0, ih0, it1, ih1,
                 u0, v0, ub0, vb0, u1, v1, ub1, vb1,
                 out_v, sem0, sem1,
                 *, n_per_w, chunk):
    wid = lax.axis_index("s") * _NC + lax.axis_index("c")
    base = wid * n_per_w
    pltpu.sync_copy(tok_hbm.at[pl.ds(base, n_per_w)], tok_v)
    pltpu.sync_copy(head_hbm.at[pl.ds(base, n_per_w)], head_v)

    nchunks = n_per_w // chunk
    bufs = ((it0, ih0, u0, v0, ub0, vb0, sem0),
            (it1, ih1, u1, v1, ub1, vb1, sem1))

    def start(g, buf):
        it, ih, u_r, v_r, ub_r, vb_r, sem = buf
        s = g * chunk

        def fill(i, _):
            sl = pl.ds(i * _L, _L)
            it[sl] = lax.shift_right_logical(tok_v[pl.ds(s + i * _L, _L)], 2)
            ih[sl] = lax.shift_right_logical(head_v[pl.ds(s + i * _L, _L)], 2)
            return 0

        lax.fori_loop(0, chunk // _L, fill, 0)
        pltpu.async_copy(u2_hbm.at[it], u_r, sem)
        pltpu.async_copy(v2_hbm.at[ih], v_r, sem)
        pltpu.async_copy(ub_hbm.at[tok_v.at[pl.ds(s, chunk)]], ub_r, sem)
        pltpu.async_copy(vb_hbm.at[head_v.at[pl.ds(s, chunk)]], vb_r, sem)

    def drain(buf):
        it, ih, u_r, v_r, ub_r, vb_r, sem = buf
        pltpu.make_async_copy(u2_hbm.at[pl.ds(0, chunk)], u_r, sem).wait()
        pltpu.make_async_copy(v2_hbm.at[pl.ds(0, chunk)], v_r, sem).wait()
        pltpu.make_async_copy(ub_hbm.at[pl.ds(0, chunk)], ub_r, sem).wait()
        pltpu.make_async_copy(vb_hbm.at[pl.ds(0, chunk)], vb_r, sem).wait()

    def compute(g, buf, accs):
        it, ih, u_r, v_r, ub_r, vb_r, _ = buf
        s = g * chunk
        three = jnp.int32(3)

        def grp(i, a):
            a0, a1, a2, a3 = a
            tvec = lax.shift_left(
                jax.lax.bitwise_and(tok_v[pl.ds(s + i * _L, _L)], three), 5)
            hvec = lax.shift_left(
                jax.lax.bitwise_and(head_v[pl.ds(s + i * _L, _L)], three), 5)
            r = i * _L
            for j in range(_L):
                cu = tvec[j]
                cv = hvec[j]
                a0 = a0 + u_r[r + j, pl.ds(cu, _L)] * v_r[r + j, pl.ds(cv, _L)]
                a1 = a1 + (u_r[r + j, pl.ds(cu + _L, _L)]
                           * v_r[r + j, pl.ds(cv + _L, _L)])
            a2 = a2 + ub_r[pl.ds(i * _L, _L)]
            a3 = a3 + vb_r[pl.ds(i * _L, _L)]
            return (a0, a1, a2, a3)

        return lax.fori_loop(0, chunk // _L, grp, accs)

    start(0, bufs[0])
    zeros = jnp.zeros((_L,), jnp.float32)

    def outer(t, accs):
        g = t * 2
        start(g + 1, bufs[1])
        drain(bufs[0])
        accs = compute(g, bufs[0], accs)

        @pl.when(g + 2 < nchunks)
        def _():
            start(g + 2, bufs[0])

        drain(bufs[1])
        return compute(g + 1, bufs[1], accs)

    accs = lax.fori_loop(0, nchunks // 2, outer, (zeros, zeros, zeros, zeros))
    out_v[...] = accs[0] + accs[1] + accs[2] + accs[3]
    pltpu.sync_copy(out_v, out_hbm.at[wid])


def kernel(tokens_batch, heads_batch, U, Ubias, V, Vbias):
    vocab = U.shape[0]
    tok = tokens_batch.reshape(-1).astype(jnp.int32)
    head = heads_batch.reshape(-1).astype(jnp.int32)
    ub = Ubias.reshape(-1)
    vb = Vbias.reshape(-1)
    n = tok.shape[0]
    n_per_w = n // _NW
    chunk = 128

    mesh = plsc.VectorSubcoreMesh(core_axis_name="c", subcore_axis_name="s")

    u2, v2 = pl.kernel(
        _repack_body,
        out_type=(jax.ShapeDtypeStruct((vocab // 4, 128), jnp.float32),
                  jax.ShapeDtypeStruct((vocab // 4, 128), jnp.float32)),
        mesh=mesh,
        scratch_types=[
            pltpu.VMEM((_RCHUNK, _DIM), jnp.float32),
            pltpu.VMEM((_RCHUNK // 4, 128), jnp.float32),
            pltpu.VMEM((_RCHUNK, _DIM), jnp.float32),
            pltpu.VMEM((_RCHUNK // 4, 128), jnp.float32),
            pltpu.SemaphoreType.DMA,
            pltpu.SemaphoreType.DMA,
        ],
        compiler_params=pltpu.CompilerParams(use_tc_tiling_on_sc=True),
    )(U, V)

    body = functools.partial(_gather_body, n_per_w=n_per_w, chunk=chunk)
    partials = pl.kernel(
        body,
        out_type=jax.ShapeDtypeStruct((_NW, _L), jnp.float32),
        mesh=mesh,
        scratch_types=[
            pltpu.VMEM((n_per_w,), jnp.int32),
            pltpu.VMEM((n_per_w,), jnp.int32),
            pltpu.VMEM((chunk,), jnp.int32),
            pltpu.VMEM((chunk,), jnp.int32),
            pltpu.VMEM((chunk,), jnp.int32),
            pltpu.VMEM((chunk,), jnp.int32),
            pltpu.VMEM((chunk, 128), jnp.float32),
            pltpu.VMEM((chunk, 128), jnp.float32),
            pltpu.VMEM((chunk,), jnp.float32),
            pltpu.VMEM((chunk,), jnp.float32),
            pltpu.VMEM((chunk, 128), jnp.float32),
            pltpu.VMEM((chunk, 128), jnp.float32),
            pltpu.VMEM((chunk,), jnp.float32),
            pltpu.VMEM((chunk,), jnp.float32),
            pltpu.VMEM((_L,), jnp.float32),
            pltpu.SemaphoreType.DMA,
            pltpu.SemaphoreType.DMA,
        ],
        compiler_params=pltpu.CompilerParams(use_tc_tiling_on_sc=False),
    )(tok, head, u2, v2, ub, vb)
    return jnp.sum(partials)


# speedup vs baseline: 1.2252x; 1.2252x over previous
import functools

import jax
import jax.numpy as jnp
from jax import lax
from jax.experimental import pallas as pl
from jax.experimental.pallas import tpu as pltpu
from jax.experimental.pallas import tpu_sc as plsc

_DIM = 32
_NC = 2
_NS = 16
_NW = _NC * _NS
_L = 16


def _sc_body(tok_hbm, head_hbm, u_hbm, v_hbm, ub_hbm, vb_hbm, out_hbm,
             tok_v, head_v,
             u0, v0, ub0, vb0, u1, v1, ub1, vb1,
             out_v, sem0, sem1,
             *, n_per_w, chunk):
    wid = lax.axis_index("s") * _NC + lax.axis_index("c")
    base = wid * n_per_w
    pltpu.sync_copy(tok_hbm.at[pl.ds(base, n_per_w)], tok_v)
    pltpu.sync_copy(head_hbm.at[pl.ds(base, n_per_w)], head_v)

    nchunks = n_per_w // chunk
    bufs = ((u0, v0, ub0, vb0, sem0), (u1, v1, ub1, vb1, sem1))

    def start(g, buf):
        u_r, v_r, ub_r, vb_r, sem = buf
        idx_t = tok_v.at[pl.ds(g * chunk, chunk)]
        idx_h = head_v.at[pl.ds(g * chunk, chunk)]
        pltpu.async_copy(u_hbm.at[idx_t], u_r, sem)
        pltpu.async_copy(v_hbm.at[idx_h], v_r, sem)
        pltpu.async_copy(ub_hbm.at[idx_t], ub_r, sem)
        pltpu.async_copy(vb_hbm.at[idx_h], vb_r, sem)

    def drain(buf):
        u_r, v_r, ub_r, vb_r, sem = buf
        pltpu.make_async_copy(u_hbm.at[pl.ds(0, chunk)], u_r, sem).wait()
        pltpu.make_async_copy(v_hbm.at[pl.ds(0, chunk)], v_r, sem).wait()
        pltpu.make_async_copy(ub_hbm.at[pl.ds(0, chunk)], ub_r, sem).wait()
        pltpu.make_async_copy(vb_hbm.at[pl.ds(0, chunk)], vb_r, sem).wait()

    def compute(buf, accs):
        u_r, v_r, ub_r, vb_r, _ = buf

        def pair4(i, a):
            a0, a1, a2, a3 = a
            r = i * 4

            def fma(a_lo, a_hi, row):
                ulo, uhi = plsc.unpack(u_r[row, :], format=plsc.PackFormat.INTERLEAVED)
                vlo, vhi = plsc.unpack(v_r[row, :], format=plsc.PackFormat.INTERLEAVED)
                return a_lo + ulo * vlo, a_hi + uhi * vhi

            a0, a1 = fma(a0, a1, r)
            a2, a3 = fma(a2, a3, r + 1)
            a0, a1 = fma(a0, a1, r + 2)
            a2, a3 = fma(a2, a3, r + 3)
            return (a0, a1, a2, a3)

        accs = lax.fori_loop(0, chunk // 4, pair4, accs)

        def bias4(k, a):
            a0, a1, a2, a3 = a
            s = k * 4 * _L
            a0 = a0 + ub_r[pl.ds(s, _L)]
            a1 = a1 + vb_r[pl.ds(s, _L)]
            a2 = a2 + ub_r[pl.ds(s + _L, _L)]
            a3 = a3 + vb_r[pl.ds(s + _L, _L)]
            a0 = a0 + ub_r[pl.ds(s + 2 * _L, _L)]
            a1 = a1 + vb_r[pl.ds(s + 2 * _L, _L)]
            a2 = a2 + ub_r[pl.ds(s + 3 * _L, _L)]
            a3 = a3 + vb_r[pl.ds(s + 3 * _L, _L)]
            return (a0, a1, a2, a3)

        return lax.fori_loop(0, chunk // (4 * _L), bias4, accs)

    start(0, bufs[0])
    zeros = jnp.zeros((_L,), jnp.float32)

    def outer(t, accs):
        g = t * 2
        start(g + 1, bufs[1])
        drain(bufs[0])
        accs = compute(bufs[0], accs)

        @pl.when(g + 2 < nchunks)
        def _():
            start(g + 2, bufs[0])

        drain(bufs[1])
        return compute(bufs[1], accs)

    accs = lax.fori_loop(0, nchunks // 2, outer, (zeros, zeros, zeros, zeros))
    out_v[...] = accs[0] + accs[1] + accs[2] + accs[3]
    pltpu.sync_copy(out_v, out_hbm.at[wid])


def kernel(tokens_batch, heads_batch, U, Ubias, V, Vbias):
    tok = tokens_batch.reshape(-1).astype(jnp.int32)
    head = heads_batch.reshape(-1).astype(jnp.int32)
    U = U.astype(jnp.bfloat16)
    V = V.astype(jnp.bfloat16)
    ub = Ubias.reshape(-1)
    vb = Vbias.reshape(-1)
    n = tok.shape[0]
    n_per_w = n // _NW
    chunk = 512

    mesh = plsc.VectorSubcoreMesh(core_axis_name="c", subcore_axis_name="s")
    body = functools.partial(_sc_body, n_per_w=n_per_w, chunk=chunk)
    partials = pl.kernel(
        body,
        out_type=jax.ShapeDtypeStruct((_NW, _L), jnp.float32),
        mesh=mesh,
        scratch_types=[
            pltpu.VMEM((n_per_w,), jnp.int32),
            pltpu.VMEM((n_per_w,), jnp.int32),
            pltpu.VMEM((chunk, _DIM), jnp.bfloat16),
            pltpu.VMEM((chunk, _DIM), jnp.bfloat16),
            pltpu.VMEM((chunk,), jnp.float32),
            pltpu.VMEM((chunk,), jnp.float32),
            pltpu.VMEM((chunk, _DIM), jnp.bfloat16),
            pltpu.VMEM((chunk, _DIM), jnp.bfloat16),
            pltpu.VMEM((chunk,), jnp.float32),
            pltpu.VMEM((chunk,), jnp.float32),
            pltpu.VMEM((_L,), jnp.float32),
            pltpu.SemaphoreType.DMA,
            pltpu.SemaphoreType.DMA,
        ],
        compiler_params=pltpu.CompilerParams(
            use_tc_tiling_on_sc=False, needs_layout_passes=False),
    )(tok, head, U, V, ub, vb)
    return jnp.sum(partials)


# speedup vs baseline: 1.5206x; 1.2411x over previous
import functools

import jax
import jax.numpy as jnp
from jax import lax
from jax.experimental import pallas as pl
from jax.experimental.pallas import tpu as pltpu
from jax.experimental.pallas import tpu_sc as plsc

_DIM = 32
_NC = 2
_NS = 16
_NW = _NC * _NS
_L = 16


def _sc_body(tok_hbm, head_hbm, u_hbm, v_hbm, ub_hbm, vb_hbm, out_hbm,
             tok_v, head_v,
             u0, v0, ub0, vb0, u1, v1, ub1, vb1,
             out_v, sem0, sem1,
             *, n_per_w, chunk):
    wid = lax.axis_index("s") * _NC + lax.axis_index("c")
    base = wid * n_per_w
    pltpu.sync_copy(tok_hbm.at[pl.ds(base, n_per_w)], tok_v)
    pltpu.sync_copy(head_hbm.at[pl.ds(base, n_per_w)], head_v)

    nchunks = n_per_w // chunk
    bufs = ((u0, v0, ub0, vb0, sem0), (u1, v1, ub1, vb1, sem1))

    def start(g, buf):
        u_r, v_r, ub_r, vb_r, sem = buf
        idx_t = tok_v.at[pl.ds(g * chunk, chunk)]
        idx_h = head_v.at[pl.ds(g * chunk, chunk)]
        pltpu.async_copy(u_hbm.at[idx_t], u_r, sem)
        pltpu.async_copy(v_hbm.at[idx_h], v_r, sem)
        pltpu.async_copy(ub_hbm.at[idx_t], ub_r, sem)
        pltpu.async_copy(vb_hbm.at[idx_h], vb_r, sem)

    def drain(buf):
        u_r, v_r, ub_r, vb_r, sem = buf
        pltpu.make_async_copy(u_hbm.at[pl.ds(0, chunk)], u_r, sem).wait()
        pltpu.make_async_copy(v_hbm.at[pl.ds(0, chunk)], v_r, sem).wait()
        pltpu.make_async_copy(ub_hbm.at[pl.ds(0, chunk)], ub_r, sem).wait()
        pltpu.make_async_copy(vb_hbm.at[pl.ds(0, chunk)], vb_r, sem).wait()

    def compute(buf, accs):
        u_r, v_r, ub_r, vb_r, _ = buf

        def pair4(i, a):
            a0, a1, a2, a3 = a
            r = i * 4
            a0 = a0 + u_r[r, pl.ds(0, _L)] * v_r[r, pl.ds(0, _L)]
            a1 = a1 + u_r[r, pl.ds(_L, _L)] * v_r[r, pl.ds(_L, _L)]
            a2 = a2 + u_r[r + 1, pl.ds(0, _L)] * v_r[r + 1, pl.ds(0, _L)]
            a3 = a3 + u_r[r + 1, pl.ds(_L, _L)] * v_r[r + 1, pl.ds(_L, _L)]
            a0 = a0 + u_r[r + 2, pl.ds(0, _L)] * v_r[r + 2, pl.ds(0, _L)]
            a1 = a1 + u_r[r + 2, pl.ds(_L, _L)] * v_r[r + 2, pl.ds(_L, _L)]
            a2 = a2 + u_r[r + 3, pl.ds(0, _L)] * v_r[r + 3, pl.ds(0, _L)]
            a3 = a3 + u_r[r + 3, pl.ds(_L, _L)] * v_r[r + 3, pl.ds(_L, _L)]
            return (a0, a1, a2, a3)

        accs = lax.fori_loop(0, chunk // 4, pair4, accs)

        def bias4(k, a):
            a0, a1, a2, a3 = a
            s = k * 4 * _L
            a0 = a0 + ub_r[pl.ds(s, _L)]
            a1 = a1 + vb_r[pl.ds(s, _L)]
            a2 = a2 + ub_r[pl.ds(s + _L, _L)]
            a3 = a3 + vb_r[pl.ds(s + _L, _L)]
            a0 = a0 + ub_r[pl.ds(s + 2 * _L, _L)]
            a1 = a1 + vb_r[pl.ds(s + 2 * _L, _L)]
            a2 = a2 + ub_r[pl.ds(s + 3 * _L, _L)]
            a3 = a3 + vb_r[pl.ds(s + 3 * _L, _L)]
            return (a0, a1, a2, a3)

        return lax.fori_loop(0, chunk // (4 * _L), bias4, accs)

    start(0, bufs[0])
    zeros = jnp.zeros((_L,), jnp.float32)

    def outer(t, accs):
        g = t * 2
        start(g + 1, bufs[1])
        drain(bufs[0])
        accs = compute(bufs[0], accs)

        @pl.when(g + 2 < nchunks)
        def _():
            start(g + 2, bufs[0])

        drain(bufs[1])
        return compute(bufs[1], accs)

    accs = lax.fori_loop(0, nchunks // 2, outer, (zeros, zeros, zeros, zeros))
    out_v[...] = accs[0] + accs[1] + accs[2] + accs[3]
    pltpu.sync_copy(out_v, out_hbm.at[wid])


def kernel(tokens_batch, heads_batch, U, Ubias, V, Vbias):
    tok = tokens_batch.reshape(-1).astype(jnp.int32)
    head = heads_batch.reshape(-1).astype(jnp.int32)
    ub = Ubias.reshape(-1)
    vb = Vbias.reshape(-1)
    n = tok.shape[0]
    n_per_w = n // _NW
    chunk = 512

    mesh = plsc.VectorSubcoreMesh(core_axis_name="c", subcore_axis_name="s")
    body = functools.partial(_sc_body, n_per_w=n_per_w, chunk=chunk)
    partials = pl.kernel(
        body,
        out_type=jax.ShapeDtypeStruct((_NW, _L), jnp.float32),
        mesh=mesh,
        scratch_types=[
            pltpu.VMEM((n_per_w,), jnp.int32),
            pltpu.VMEM((n_per_w,), jnp.int32),
            pltpu.VMEM((chunk, _DIM), jnp.float32),
            pltpu.VMEM((chunk, _DIM), jnp.float32),
            pltpu.VMEM((chunk,), jnp.float32),
            pltpu.VMEM((chunk,), jnp.float32),
            pltpu.VMEM((chunk, _DIM), jnp.float32),
            pltpu.VMEM((chunk, _DIM), jnp.float32),
            pltpu.VMEM((chunk,), jnp.float32),
            pltpu.VMEM((chunk,), jnp.float32),
            pltpu.VMEM((_L,), jnp.float32),
            pltpu.SemaphoreType.DMA,
            pltpu.SemaphoreType.DMA,
        ],
        compiler_params=pltpu.CompilerParams(use_tc_tiling_on_sc=False),
    )(tok, head, U, V, ub, vb)
    return jnp.sum(partials)
